# baseline (speedup 1.0000x reference)
import jax
import jax.numpy as jnp
from jax import lax
from jax.experimental import pallas as pl
from jax.experimental.pallas import tpu as pltpu
from jax.experimental.pallas import tpu_sc as plsc

_B, _S, _D = 4, 8192, 768
_NC, _NS = 2, 16
_NW = _NC * _NS
_SCBATCH = 3
_RPW = _S // _NW
_CHR = 16
_NCH = _RPW // _CHR


def _sc_body(x_hbm, pe_hbm, o_hbm, peb, xbuf, insems, outsems):
    wid = lax.axis_index("s") * _NC + lax.axis_index("c")
    row0 = wid * _RPW

    in_cps = [None, None]
    out_cps = [None, None]

    def start_in(c):
        slot = c & 1
        r = row0 + c * _CHR
        cp_pe = pltpu.async_copy(
            pe_hbm.at[pl.ds(r, _CHR)], peb.at[slot], insems.at[slot]
        )
        cp_x = pltpu.async_copy(
            x_hbm.at[_SCBATCH, pl.ds(r, _CHR)], xbuf.at[slot], insems.at[slot]
        )
        in_cps[slot] = (cp_pe, cp_x)

    start_in(0)
    for c in range(_NCH):
        slot = c & 1
        if c + 1 < _NCH:
            nslot = (c + 1) & 1
            if out_cps[nslot] is not None:
                out_cps[nslot].wait()
                out_cps[nslot] = None
            start_in(c + 1)
        cp_pe, cp_x = in_cps[slot]
        cp_pe.wait()
        cp_x.wait()

        @pl.loop(0, _CHR)
        def _row(rr, _s=slot):
            @plsc.parallel_loop(0, _D, step=16, unroll=8)
            def _(j):
                sl = pl.ds(j, 16)
                xbuf[_s, rr, sl] = xbuf[_s, rr, sl] + peb[_s, rr, sl]

        out_cps[slot] = pltpu.async_copy(
            xbuf.at[slot],
            o_hbm.at[pl.ds(row0 + c * _CHR, _CHR)],
            outsems.at[slot],
        )
    for oc in out_cps:
        if oc is not None:
            oc.wait()


def _sc_part(x, pe_table):
    mesh = plsc.VectorSubcoreMesh(core_axis_name="c", subcore_axis_name="s")
    run = pl.kernel(
        _sc_body,
        out_type=jax.ShapeDtypeStruct((_S, _D), jnp.float32),
        mesh=mesh,
        scratch_types=[
            pltpu.VMEM((2, _CHR, _D), jnp.float32),
            pltpu.VMEM((2, _CHR, _D), jnp.float32),
            pltpu.SemaphoreType.DMA((2,)),
            pltpu.SemaphoreType.DMA((2,)),
        ],
    )
    return run(x, pe_table)


_BS = 512


def _tc1_body(x_ref, pe_ref, o_ref):
    o_ref[...] = x_ref[...] + pe_ref[...]


def _tc1_part(x, pe_table):
    grid = (_S // _BS, 2)
    return pl.pallas_call(
        _tc1_body,
        grid=grid,
        in_specs=[
            pl.BlockSpec((1, _BS, _D), lambda s, b: (b, s, 0)),
            pl.BlockSpec((_BS, _D), lambda s, b: (s, 0)),
        ],
        out_specs=pl.BlockSpec((1, _BS, _D), lambda s, b: (b, s, 0)),
        out_shape=jax.ShapeDtypeStruct((_B, _S, _D), jnp.float32),
    )(x, pe_table)


def _tc2_body(tc_ref, x_ref, pe_ref, sc_ref, o_ref):
    del tc_ref
    o_ref[0] = x_ref[0] + pe_ref[...]
    o_ref[1] = sc_ref[...]


def _tc2_part(tc_full, x, pe_table, sc_out):
    return pl.pallas_call(
        _tc2_body,
        grid=(_S // _BS,),
        in_specs=[
            pl.BlockSpec(memory_space=pl.ANY),
            pl.BlockSpec((1, _BS, _D), lambda s: (2, s, 0)),
            pl.BlockSpec((_BS, _D), lambda s: (s, 0)),
            pl.BlockSpec((_BS, _D), lambda s: (s, 0)),
        ],
        out_specs=pl.BlockSpec((2, _BS, _D), lambda s: (1, s, 0)),
        out_shape=jax.ShapeDtypeStruct((_B, _S, _D), jnp.float32),
        input_output_aliases={0: 0},
    )(tc_full, x, pe_table, sc_out)


def kernel(x, pe_table):
    sc_out = _sc_part(x, pe_table)
    tc_full = _tc1_part(x, pe_table)
    return _tc2_part(tc_full, x, pe_table, sc_out)


# speedup vs baseline: 1.7880x; 1.7880x over previous
import jax
import jax.numpy as jnp
from jax import lax
from jax.experimental import pallas as pl
from jax.experimental.pallas import tpu as pltpu
from jax.experimental.pallas import tpu_sc as plsc

_B, _S, _D = 4, 8192, 768
_NC, _NS = 2, 16
_NW = _NC * _NS
_SCBATCH = 3
_RPW = _S // _NW
_CHR = 16
_NCH = _RPW // _CHR


def _sc_body(x_hbm, pe_hbm, o_hbm, peb, xbuf, insems, outsems):
    wid = lax.axis_index("s") * _NC + lax.axis_index("c")
    row0 = wid * _RPW

    in_cps = [None, None]
    out_cps = [None, None]

    def start_in(c):
        slot = c & 1
        r = row0 + c * _CHR
        cp_pe = pltpu.async_copy(
            pe_hbm.at[pl.ds(r, _CHR)], peb.at[slot], insems.at[slot]
        )
        cp_x = pltpu.async_copy(
            x_hbm.at[_SCBATCH, pl.ds(r, _CHR)], xbuf.at[slot], insems.at[slot]
        )
        in_cps[slot] = (cp_pe, cp_x)

    start_in(0)
    for c in range(_NCH):
        slot = c & 1
        if c + 1 < _NCH:
            nslot = (c + 1) & 1
            if out_cps[nslot] is not None:
                out_cps[nslot].wait()
                out_cps[nslot] = None
            start_in(c + 1)
        cp_pe, cp_x = in_cps[slot]
        cp_pe.wait()
        cp_x.wait()

        @pl.loop(0, _CHR)
        def _row(rr, _s=slot):
            @plsc.parallel_loop(0, _D, step=16, unroll=8)
            def _(j):
                sl = pl.ds(j, 16)
                xbuf[_s, rr, sl] = xbuf[_s, rr, sl] + peb[_s, rr, sl]

        out_cps[slot] = pltpu.async_copy(
            xbuf.at[slot],
            o_hbm.at[pl.ds(row0 + c * _CHR, _CHR)],
            outsems.at[slot],
        )
    for oc in out_cps:
        if oc is not None:
            oc.wait()


def _sc_part(x, pe_table):
    mesh = plsc.VectorSubcoreMesh(core_axis_name="c", subcore_axis_name="s")
    run = pl.kernel(
        _sc_body,
        out_type=jax.ShapeDtypeStruct((_S, _D), jnp.float32),
        mesh=mesh,
        scratch_types=[
            pltpu.VMEM((2, _CHR, _D), jnp.float32),
            pltpu.VMEM((2, _CHR, _D), jnp.float32),
            pltpu.SemaphoreType.DMA((2,)),
            pltpu.SemaphoreType.DMA((2,)),
        ],
    )
    return run(x, pe_table)


_BS = 512


def _tc_body(x_ref, pe_ref, o_ref):
    o_ref[...] = x_ref[...] + pe_ref[...]


def _tc_part(x, pe_table):
    grid = (_S // _BS, _SCBATCH)
    return pl.pallas_call(
        _tc_body,
        grid=grid,
        in_specs=[
            pl.BlockSpec((1, _BS, _D), lambda s, b: (b, s, 0)),
            pl.BlockSpec((_BS, _D), lambda s, b: (s, 0)),
        ],
        out_specs=pl.BlockSpec((1, _BS, _D), lambda s, b: (b, s, 0)),
        out_shape=jax.ShapeDtypeStruct((_B, _S, _D), jnp.float32),
    )(x, pe_table)


def _merge_body(tc_ref, sc_ref, o_ref):
    del tc_ref
    o_ref[...] = sc_ref[...][None]


def _merge(tc_full, sc_out):
    return pl.pallas_call(
        _merge_body,
        grid=(_S // _BS,),
        in_specs=[
            pl.BlockSpec(memory_space=pl.ANY),
            pl.BlockSpec((_BS, _D), lambda s: (s, 0)),
        ],
        out_specs=pl.BlockSpec((1, _BS, _D), lambda s: (_SCBATCH, s, 0)),
        out_shape=jax.ShapeDtypeStruct((_B, _S, _D), jnp.float32),
        input_output_aliases={0: 0},
    )(tc_full, sc_out)


def _tc_wide_body(x_ref, pe_ref, o_ref):
    o_ref[...] = x_ref[...] + pe_ref[...][None]


def kernel(x, pe_table):
    return pl.pallas_call(
        _tc_wide_body,
        grid=(_S // _BS,),
        in_specs=[
            pl.BlockSpec((_B, _BS, _D), lambda s: (0, s, 0)),
            pl.BlockSpec((_BS, _D), lambda s: (s, 0)),
        ],
        out_specs=pl.BlockSpec((_B, _BS, _D), lambda s: (0, s, 0)),
        out_shape=jax.ShapeDtypeStruct((_B, _S, _D), jnp.float32),
    )(x, pe_table)


# speedup vs baseline: 1.7940x; 1.0034x over previous
import jax
import jax.numpy as jnp
from jax import lax
from jax.experimental import pallas as pl
from jax.experimental.pallas import tpu as pltpu
from jax.experimental.pallas import tpu_sc as plsc

_B, _S, _D = 4, 8192, 768
_NC, _NS = 2, 16
_NW = _NC * _NS
_SCBATCH = 3
_RPW = _S // _NW
_CHR = 16
_NCH = _RPW // _CHR


def _sc_body(x_hbm, pe_hbm, o_hbm, peb, xbuf, insems, outsems):
    wid = lax.axis_index("s") * _NC + lax.axis_index("c")
    row0 = wid * _RPW

    in_cps = [None, None]
    out_cps = [None, None]

    def start_in(c):
        slot = c & 1
        r = row0 + c * _CHR
        cp_pe = pltpu.async_copy(
            pe_hbm.at[pl.ds(r, _CHR)], peb.at[slot], insems.at[slot]
        )
        cp_x = pltpu.async_copy(
            x_hbm.at[_SCBATCH, pl.ds(r, _CHR)], xbuf.at[slot], insems.at[slot]
        )
        in_cps[slot] = (cp_pe, cp_x)

    start_in(0)
    for c in range(_NCH):
        slot = c & 1
        if c + 1 < _NCH:
            nslot = (c + 1) & 1
            if out_cps[nslot] is not None:
                out_cps[nslot].wait()
                out_cps[nslot] = None
            start_in(c + 1)
        cp_pe, cp_x = in_cps[slot]
        cp_pe.wait()
        cp_x.wait()

        @pl.loop(0, _CHR)
        def _row(rr, _s=slot):
            @plsc.parallel_loop(0, _D, step=16, unroll=8)
            def _(j):
                sl = pl.ds(j, 16)
                xbuf[_s, rr, sl] = xbuf[_s, rr, sl] + peb[_s, rr, sl]

        out_cps[slot] = pltpu.async_copy(
            xbuf.at[slot],
            o_hbm.at[pl.ds(row0 + c * _CHR, _CHR)],
            outsems.at[slot],
        )
    for oc in out_cps:
        if oc is not None:
            oc.wait()


def _sc_part(x, pe_table):
    mesh = plsc.VectorSubcoreMesh(core_axis_name="c", subcore_axis_name="s")
    run = pl.kernel(
        _sc_body,
        out_type=jax.ShapeDtypeStruct((_S, _D), jnp.float32),
        mesh=mesh,
        scratch_types=[
            pltpu.VMEM((2, _CHR, _D), jnp.float32),
            pltpu.VMEM((2, _CHR, _D), jnp.float32),
            pltpu.SemaphoreType.DMA((2,)),
            pltpu.SemaphoreType.DMA((2,)),
        ],
    )
    return run(x, pe_table)


_BS = 1024


def _tc_body(x_ref, pe_ref, o_ref):
    o_ref[...] = x_ref[...] + pe_ref[...]


def _tc_part(x, pe_table):
    grid = (_S // _BS, _SCBATCH)
    return pl.pallas_call(
        _tc_body,
        grid=grid,
        in_specs=[
            pl.BlockSpec((1, _BS, _D), lambda s, b: (b, s, 0)),
            pl.BlockSpec((_BS, _D), lambda s, b: (s, 0)),
        ],
        out_specs=pl.BlockSpec((1, _BS, _D), lambda s, b: (b, s, 0)),
        out_shape=jax.ShapeDtypeStruct((_B, _S, _D), jnp.float32),
    )(x, pe_table)


def _merge_body(tc_ref, sc_ref, o_ref):
    del tc_ref
    o_ref[...] = sc_ref[...][None]


def _merge(tc_full, sc_out):
    return pl.pallas_call(
        _merge_body,
        grid=(_S // _BS,),
        in_specs=[
            pl.BlockSpec(memory_space=pl.ANY),
            pl.BlockSpec((_BS, _D), lambda s: (s, 0)),
        ],
        out_specs=pl.BlockSpec((1, _BS, _D), lambda s: (_SCBATCH, s, 0)),
        out_shape=jax.ShapeDtypeStruct((_B, _S, _D), jnp.float32),
        input_output_aliases={0: 0},
    )(tc_full, sc_out)


def _tc_wide_body(x_ref, pe_ref, o_ref):
    o_ref[...] = x_ref[...] + pe_ref[...][None]


def kernel(x, pe_table):
    return pl.pallas_call(
        _tc_wide_body,
        grid=(_S // _BS,),
        in_specs=[
            pl.BlockSpec((_B, _BS, _D), lambda s: (0, s, 0)),
            pl.BlockSpec((_BS, _D), lambda s: (s, 0)),
        ],
        out_specs=pl.BlockSpec((_B, _BS, _D), lambda s: (0, s, 0)),
        out_shape=jax.ShapeDtypeStruct((_B, _S, _D), jnp.float32),
    )(x, pe_table)
